# baseline (speedup 1.0000x reference)
import jax
import jax.numpy as jnp
from jax.experimental import pallas as pl

N0, N1, EMB = 384, 384, 96
BI = 32


def _body(e0t_ref, e1_ref, out_ref):
    out_ref[...] = e0t_ref[...] + e1_ref[...]


def kernel(x, emb_0, emb_1):
    del x
    e0t = jnp.tile(emb_0, (1, 4)).reshape(N0, 1, N1)
    e1r = emb_1.reshape(1, EMB, N1)
    out = pl.pallas_call(
        _body,
        grid=(N0 // BI,),
        in_specs=[
            pl.BlockSpec((BI, 1, N1), lambda g: (g, 0, 0)),
            pl.BlockSpec((1, EMB, N1), lambda g: (0, 0, 0)),
        ],
        out_specs=pl.BlockSpec((BI, EMB, N1), lambda g: (g, 0, 0)),
        out_shape=jax.ShapeDtypeStruct((N0, EMB, N1), jnp.float32),
    )(e0t, e1r)
    return out.reshape(1, N0, N1, EMB)


# speedup vs baseline: 1.2374x; 1.2374x over previous
import jax
import jax.numpy as jnp
from jax.experimental import pallas as pl

N0, N1, EMB = 384, 384, 96
BI = 8


def _body(e0_ref, e1_ref, out_ref):
    e1 = e1_ref[...]
    for b in range(BI):
        out_ref[b] = e1 + e0_ref[b]


def kernel(x, emb_0, emb_1):
    del x
    out = pl.pallas_call(
        _body,
        grid=(N0 // BI,),
        in_specs=[
            pl.BlockSpec((BI, EMB), lambda g: (g, 0)),
            pl.BlockSpec((N1, EMB), lambda g: (0, 0)),
        ],
        out_specs=pl.BlockSpec((BI, N1, EMB), lambda g: (g, 0, 0)),
        out_shape=jax.ShapeDtypeStruct((N0, N1, EMB), jnp.float32),
    )(emb_0, emb_1)
    return out[None]


# speedup vs baseline: 2.1772x; 1.7596x over previous
import jax
import jax.numpy as jnp
from jax.experimental import pallas as pl

N0, N1, EMB = 384, 384, 96
BI = 8


def _body(e0_ref, e1_ref, out_ref):
    e1 = e1_ref[...]
    for b in range(BI):
        out_ref[b] = e1 + e0_ref[b]


def kernel(x, emb_0, emb_1):
    del x
    e1t = emb_1.T
    e0c = emb_0[:, :, None]
    out3 = pl.pallas_call(
        _body,
        grid=(N0 // BI,),
        in_specs=[
            pl.BlockSpec((BI, EMB, 1), lambda g: (g, 0, 0)),
            pl.BlockSpec((EMB, N1), lambda g: (0, 0)),
        ],
        out_specs=pl.BlockSpec((BI, EMB, N1), lambda g: (g, 0, 0)),
        out_shape=jax.ShapeDtypeStruct((N0, EMB, N1), jnp.float32),
    )(e0c, e1t)
    return out3.transpose(0, 2, 1)[None]


# speedup vs baseline: 3.0785x; 1.4140x over previous
import jax
import jax.numpy as jnp
from jax.experimental import pallas as pl

N0, N1, EMB = 384, 384, 96
BI = 32


def _body(e0_ref, e1_ref, out_ref):
    e1 = e1_ref[...]
    for b in range(BI):
        out_ref[b] = e1 + e0_ref[b]


def kernel(x, emb_0, emb_1):
    del x
    e1t = emb_1.T
    e0c = emb_0[:, :, None]
    out3 = pl.pallas_call(
        _body,
        grid=(N0 // BI,),
        in_specs=[
            pl.BlockSpec((BI, EMB, 1), lambda g: (g, 0, 0)),
            pl.BlockSpec((EMB, N1), lambda g: (0, 0)),
        ],
        out_specs=pl.BlockSpec((BI, EMB, N1), lambda g: (g, 0, 0)),
        out_shape=jax.ShapeDtypeStruct((N0, EMB, N1), jnp.float32),
    )(e0c, e1t)
    return out3.transpose(0, 2, 1)[None]


# speedup vs baseline: 3.2872x; 1.0678x over previous
import jax
import jax.numpy as jnp
from jax.experimental import pallas as pl

N0, N1, EMB = 384, 384, 96
BI = 96


def _body(e0_ref, e1_ref, out_ref):
    e1 = e1_ref[...]
    for b in range(BI):
        out_ref[b] = e1 + e0_ref[b]


def kernel(x, emb_0, emb_1):
    del x
    e1t = emb_1.T
    e0c = emb_0[:, :, None]
    out3 = pl.pallas_call(
        _body,
        grid=(N0 // BI,),
        in_specs=[
            pl.BlockSpec((BI, EMB, 1), lambda g: (g, 0, 0)),
            pl.BlockSpec((EMB, N1), lambda g: (0, 0)),
        ],
        out_specs=pl.BlockSpec((BI, EMB, N1), lambda g: (g, 0, 0)),
        out_shape=jax.ShapeDtypeStruct((N0, EMB, N1), jnp.float32),
    )(e0c, e1t)
    return out3.transpose(0, 2, 1)[None]
